# baseline (speedup 1.0000x reference)
import functools

import jax
import jax.numpy as jnp
from jax import lax
from jax.experimental import pallas as pl
from jax.experimental.pallas import tpu as pltpu
from jax.experimental.pallas import tpu_sc as plsc

EMBED_DIM = 64
NC = 2
NS = 16
NW = NC * NS
G = 128
NBUF = 4


@functools.partial(jax.jit, static_argnums=(2,))
def _emb(idx, wpad, ng):
    b_per_w = ng * G
    mesh = plsc.VectorSubcoreMesh(
        core_axis_name="c", subcore_axis_name="s", num_cores=NC,
        num_subcores=NS)

    @functools.partial(
        pl.kernel,
        out_type=jax.ShapeDtypeStruct(
            (NW * b_per_w, 2 * EMBED_DIM), jnp.float32),
        mesh=mesh,
        scratch_types=[
            pltpu.VMEM((ng, G), jnp.int32),
            pltpu.VMEM((NBUF, G), jnp.int32),
            pltpu.VMEM((NBUF, G, 2 * EMBED_DIM), jnp.float32),
            pltpu.SemaphoreType.DMA,
            pltpu.SemaphoreType.DMA,
        ],
        compiler_params=pltpu.CompilerParams(
            use_tc_tiling_on_sc=True, needs_layout_passes=False),
    )
    def body(idx_hbm, w_hbm, out_hbm, idx_v, idxs, pbuf, sem_g, sem_w):
        w = lax.axis_index("s") * NC + lax.axis_index("c")
        base = w * b_per_w
        pltpu.sync_copy(idx_hbm.at[w], idx_v)

        def fire_g(s, b):
            for k in range(G // 16):
                idxs[b, pl.ds(16 * k, 16)] = idx_v[s, pl.ds(16 * k, 16)]
            pltpu.async_copy(w_hbm.at[idxs.at[b]], pbuf.at[b], sem_g)

        def wait_g(s, b):
            pltpu.make_async_copy(
                w_hbm.at[idxs.at[b]], pbuf.at[b], sem_g).wait()

        def fire_w(s, b):
            pltpu.async_copy(
                pbuf.at[b], out_hbm.at[pl.ds(base + s * G, G)], sem_w)

        def wait_w(s, b):
            pltpu.make_async_copy(
                pbuf.at[b], out_hbm.at[pl.ds(base + s * G, G)], sem_w).wait()

        nq = ng // NBUF
        half = NBUF // 2
        fire_g(0, 0)
        fire_g(1, 1)

        def step(q, carry):
            for b in range(NBUF):
                s = q * NBUF + b
                b2 = (b + half) % NBUF

                wait_g(s, b)
                fire_w(s, b)

                @pl.when(s >= half)
                def _():
                    wait_w(s - half, b2)

                @pl.when(s + half < ng)
                def _():
                    fire_g(s + half, b2)

            return carry

        lax.fori_loop(0, nq, step, 0)
        for b in range(half):
            s = ng - half + b
            wait_w(s, s % NBUF)

    return body(idx, wpad)


def kernel(x, weight):
    bsz, ns = x.shape
    total = bsz * ns
    ng = total // (NW * G)
    wpad = jnp.concatenate(
        [weight, jnp.zeros((weight.shape[0], 2 * EMBED_DIM - weight.shape[1]),
                           weight.dtype)], axis=1)
    idx = x.reshape(NW, ng, G).astype(jnp.int32)
    out = _emb(idx, wpad, ng)
    return out[:, :EMBED_DIM].reshape(bsz, ns, EMBED_DIM)
